# baseline (speedup 1.0000x reference)
import functools

import jax
import jax.numpy as jnp
from jax import lax
from jax.experimental import pallas as pl
from jax.experimental.pallas import tpu as pltpu
from jax.experimental.pallas import tpu_sc as plsc

NC = 2
NS = 16
NW = NC * NS
D = 128
SLAB = 8
BR = 1024
DW = 16


def _mesh():
  return plsc.VectorSubcoreMesh(
      core_axis_name="c", subcore_axis_name="s", num_cores=NC,
      num_subcores=NS)


def _make_deg_kernel(npad, ch):
  rpt = npad // NS

  @functools.partial(
      pl.kernel,
      out_type=jax.ShapeDtypeStruct((NC, npad, DW), jnp.float32),
      mesh=_mesh(),
      scratch_types=[
          pltpu.VMEM((ch, D), jnp.int32),
          pltpu.VMEM((D, DW), jnp.float32),
          pltpu.VMEM_SHARED((npad, DW), jnp.float32),
      ],
  )
  def deg_kernel(dst_hbm, out_hbm, dst_v, buf, shist):
    c = lax.axis_index("c")
    s = lax.axis_index("s")
    wid = s * NC + c

    def fill(j, val):
      buf[j, pl.ds(0, 16)] = jnp.full((16,), val, jnp.float32)
      return val

    lax.fori_loop(0, D, fill, 0.0)
    for k in range(rpt // D):
      pltpu.sync_copy(buf, shist.at[pl.ds(s * rpt + k * D, D)])
    lax.fori_loop(0, D, fill, 1.0)
    plsc.subcore_barrier()

    pltpu.sync_copy(dst_hbm.at[wid], dst_v)

    def body(j, carry):
      pltpu.sync_copy(buf, shist.at[dst_v.at[j]], add=True)
      return carry

    lax.fori_loop(0, ch, body, 0)
    plsc.subcore_barrier()

    pltpu.sync_copy(shist.at[pl.ds(s * rpt, rpt)],
                    out_hbm.at[c, pl.ds(s * rpt, rpt)])

  return deg_kernel


def _make_agg_kernel(npad, nslab):
  rpt = npad // NS

  @functools.partial(
      pl.kernel,
      out_type=jax.ShapeDtypeStruct((NC, npad, D), jnp.float32),
      mesh=_mesh(),
      scratch_types=[
          pltpu.VMEM((2, 2, SLAB, D), jnp.int32),
          pltpu.VMEM((2, D, D), jnp.float32),
          pltpu.VMEM_SHARED((npad, D), jnp.float32),
          pltpu.SemaphoreType.DMA,
          pltpu.SemaphoreType.DMA,
          pltpu.SemaphoreType.DMA,
          pltpu.SemaphoreType.DMA,
      ],
  )
  def agg_kernel(g_hbm, e_hbm, out_hbm, idx_v, rowbuf, acc,
                 semi0, semi1, semg0, semg1):
    c = lax.axis_index("c")
    s = lax.axis_index("s")
    wid = s * NC + c
    semi = (semi0, semi1)
    semg = (semg0, semg1)

    def zrow(j, carry):
      for k in range(D // 16):
        rowbuf[0, j, pl.ds(k * 16, 16)] = jnp.zeros((16,), jnp.float32)
      return carry

    lax.fori_loop(0, D, zrow, 0)
    for k in range(rpt // D):
      pltpu.sync_copy(rowbuf.at[0], acc.at[pl.ds(s * rpt + k * D, D)])
    plsc.subcore_barrier()

    pltpu.async_copy(e_hbm.at[wid, 0], idx_v.at[0], semi[0])

    def slab(t, sb):
      pltpu.make_async_copy(e_hbm.at[wid, t], idx_v.at[sb], semi[sb]).wait()

      @pl.when(t + 1 < nslab)
      def _():
        pltpu.async_copy(e_hbm.at[wid, t + 1], idx_v.at[1 - sb], semi[1 - sb])

      pltpu.async_copy(g_hbm.at[idx_v.at[sb, 0, 0]], rowbuf.at[0], semg[0])

      def chunks(kk, carry):
        for b in range(2):
          k = kk * 2 + b
          pltpu.make_async_copy(
              g_hbm.at[idx_v.at[sb, 0, k]], rowbuf.at[b], semg[b]).wait()

          @pl.when(k + 1 < SLAB)
          def _():
            pltpu.async_copy(
                g_hbm.at[idx_v.at[sb, 0, k + 1]], rowbuf.at[1 - b],
                semg[1 - b])

          pltpu.sync_copy(rowbuf.at[b], acc.at[idx_v.at[sb, 1, k]], add=True)
        return carry

      lax.fori_loop(0, SLAB // 2, chunks, 0)

    def outer(tt, carry):
      for sb in range(2):
        slab(tt * 2 + sb, sb)
      return carry

    lax.fori_loop(0, nslab // 2, outer, 0)
    plsc.subcore_barrier()

    pltpu.sync_copy(acc.at[pl.ds(s * rpt, rpt)],
                    out_hbm.at[c, pl.ds(s * rpt, rpt)])

  return agg_kernel


def _tc_pre(xp, W1, degp):
  npad = xp.shape[0]

  def body(x_ref, w_ref, dp_ref, g_ref, dinv_ref):
    deg = dp_ref[0, :, 0:1] + dp_ref[1, :, 0:1] + 1.0
    dinv = 1.0 / jnp.sqrt(deg)
    dinv_ref[...] = dinv
    g_ref[...] = jnp.dot(
        x_ref[...], w_ref[...], preferred_element_type=jnp.float32) * dinv

  return pl.pallas_call(
      body,
      grid=(npad // BR,),
      in_specs=[
          pl.BlockSpec((BR, D), lambda i: (i, 0)),
          pl.BlockSpec((D, D), lambda i: (0, 0)),
          pl.BlockSpec((NC, BR, DW), lambda i: (0, i, 0)),
      ],
      out_specs=[
          pl.BlockSpec((BR, D), lambda i: (i, 0)),
          pl.BlockSpec((BR, 1), lambda i: (i, 0)),
      ],
      out_shape=[
          jax.ShapeDtypeStruct((npad, D), jnp.float32),
          jax.ShapeDtypeStruct((npad, 1), jnp.float32),
      ],
  )(xp, W1, degp)


def _tc_mid(p, g, dinv, b, W):
  npad = g.shape[0]

  def body(p_ref, g_ref, dinv_ref, b_ref, w_ref, out_ref):
    t = (p_ref[0] + p_ref[1] + g_ref[...]) * dinv_ref[...] + b_ref[...]
    t = jnp.maximum(t, 0.0)
    out_ref[...] = jnp.dot(
        t, w_ref[...], preferred_element_type=jnp.float32) * dinv_ref[...]

  return pl.pallas_call(
      body,
      grid=(npad // BR,),
      in_specs=[
          pl.BlockSpec((NC, BR, D), lambda i: (0, i, 0)),
          pl.BlockSpec((BR, D), lambda i: (i, 0)),
          pl.BlockSpec((BR, 1), lambda i: (i, 0)),
          pl.BlockSpec((1, D), lambda i: (0, 0)),
          pl.BlockSpec((D, D), lambda i: (0, 0)),
      ],
      out_specs=pl.BlockSpec((BR, D), lambda i: (i, 0)),
      out_shape=jax.ShapeDtypeStruct((npad, D), jnp.float32),
  )(p, g, dinv, b, W)


def _tc_post(p, g, dinv, b):
  npad = g.shape[0]

  def body(p_ref, g_ref, dinv_ref, b_ref, out_ref):
    out_ref[...] = (
        (p_ref[0] + p_ref[1] + g_ref[...]) * dinv_ref[...] + b_ref[...])

  return pl.pallas_call(
      body,
      grid=(npad // BR,),
      in_specs=[
          pl.BlockSpec((NC, BR, D), lambda i: (0, i, 0)),
          pl.BlockSpec((BR, D), lambda i: (i, 0)),
          pl.BlockSpec((BR, 1), lambda i: (i, 0)),
          pl.BlockSpec((1, D), lambda i: (0, 0)),
      ],
      out_specs=pl.BlockSpec((BR, D), lambda i: (i, 0)),
      out_shape=jax.ShapeDtypeStruct((npad, D), jnp.float32),
  )(p, g, dinv, b)


def kernel(x, edge_index, W1, b1, W2, b2, W3, b3):
  N, d_in = x.shape
  E = edge_index.shape[1]

  npad = -(-(N + 1) // (NS * D)) * (NS * D)
  ch = -(-E // (NW * D * 2 * SLAB)) * (2 * SLAB)
  nslab = ch // SLAB
  epad = NW * ch * D

  src = edge_index[0].astype(jnp.int32)
  dst = edge_index[1].astype(jnp.int32)
  pad = jnp.full((epad - E,), N, jnp.int32)
  src5 = jnp.concatenate([src, pad]).reshape(NW, nslab, 1, SLAB, D)
  dst5 = jnp.concatenate([dst, pad]).reshape(NW, nslab, 1, SLAB, D)
  edges5 = jnp.concatenate([src5, dst5], axis=2)
  dst3 = dst5.reshape(NW, ch, D)
  xp = jnp.concatenate([x, jnp.zeros((npad - N, d_in), x.dtype)])

  degp = _make_deg_kernel(npad, ch)(dst3)

  agg = _make_agg_kernel(npad, nslab)

  g, dinv = _tc_pre(xp, W1, degp)
  p = agg(g, edges5)
  g = _tc_mid(p, g, dinv, b1.reshape(1, D), W2)
  p = agg(g, edges5)
  g = _tc_mid(p, g, dinv, b2.reshape(1, D), W3)
  p = agg(g, edges5)
  out = _tc_post(p, g, dinv, b3.reshape(1, D))
  return out[:N]


# speedup vs baseline: 1.0003x; 1.0003x over previous
import functools

import jax
import jax.numpy as jnp
from jax import lax
from jax.experimental import pallas as pl
from jax.experimental.pallas import tpu as pltpu
from jax.experimental.pallas import tpu_sc as plsc

NC = 2
NS = 16
NW = NC * NS
D = 128
SLAB = 8
BR = 1024
DW = 16


def _mesh():
  return plsc.VectorSubcoreMesh(
      core_axis_name="c", subcore_axis_name="s", num_cores=NC,
      num_subcores=NS)


def _make_deg_kernel(npad, ch):
  rpt = npad // NS

  @functools.partial(
      pl.kernel,
      out_type=jax.ShapeDtypeStruct((NC, npad, DW), jnp.float32),
      mesh=_mesh(),
      scratch_types=[
          pltpu.VMEM((ch, D), jnp.int32),
          pltpu.VMEM((D, DW), jnp.float32),
          pltpu.VMEM_SHARED((npad, DW), jnp.float32),
      ],
  )
  def deg_kernel(dst_hbm, out_hbm, dst_v, buf, shist):
    c = lax.axis_index("c")
    s = lax.axis_index("s")
    wid = s * NC + c

    def fill(j, val):
      buf[j, pl.ds(0, 16)] = jnp.full((16,), val, jnp.float32)
      return val

    lax.fori_loop(0, D, fill, 0.0)
    for k in range(rpt // D):
      pltpu.sync_copy(buf, shist.at[pl.ds(s * rpt + k * D, D)])
    lax.fori_loop(0, D, fill, 1.0)
    plsc.subcore_barrier()

    pltpu.sync_copy(dst_hbm.at[wid], dst_v)

    def body(j, carry):
      pltpu.sync_copy(buf, shist.at[dst_v.at[j]], add=True)
      return carry

    lax.fori_loop(0, ch, body, 0)
    plsc.subcore_barrier()

    pltpu.sync_copy(shist.at[pl.ds(s * rpt, rpt)],
                    out_hbm.at[c, pl.ds(s * rpt, rpt)])

  return deg_kernel


def _make_agg_kernel(npad, nslab):
  rpt = npad // NS

  @functools.partial(
      pl.kernel,
      out_type=jax.ShapeDtypeStruct((NC, npad, D), jnp.float32),
      mesh=_mesh(),
      scratch_types=[
          pltpu.VMEM((2, 2, SLAB, D), jnp.int32),
          pltpu.VMEM((2, D, D), jnp.float32),
          pltpu.VMEM_SHARED((npad, D), jnp.float32),
          pltpu.SemaphoreType.DMA,
          pltpu.SemaphoreType.DMA,
          pltpu.SemaphoreType.DMA,
          pltpu.SemaphoreType.DMA,
      ],
  )
  def agg_kernel(g_hbm, e_hbm, out_hbm, idx_v, rowbuf, acc,
                 semi0, semi1, semg0, semg1):
    c = lax.axis_index("c")
    s = lax.axis_index("s")
    wid = s * NC + c
    semi = (semi0, semi1)
    semg = (semg0, semg1)

    def zrow(j, carry):
      for k in range(D // 16):
        rowbuf[0, j, pl.ds(k * 16, 16)] = jnp.zeros((16,), jnp.float32)
      return carry

    lax.fori_loop(0, D, zrow, 0)
    for k in range(rpt // D):
      pltpu.sync_copy(rowbuf.at[0], acc.at[pl.ds(s * rpt + k * D, D)])
    plsc.subcore_barrier()

    pltpu.async_copy(e_hbm.at[wid, 0], idx_v.at[0], semi[0])

    def slab(t, sb):
      pltpu.make_async_copy(e_hbm.at[wid, t], idx_v.at[sb], semi[sb]).wait()

      @pl.when(t + 1 < nslab)
      def _():
        pltpu.async_copy(e_hbm.at[wid, t + 1], idx_v.at[1 - sb], semi[1 - sb])

      pltpu.async_copy(g_hbm.at[idx_v.at[sb, 0, 0]], rowbuf.at[0], semg[0])

      def chunks(kk, carry):
        for b in range(2):
          k = kk * 2 + b
          pltpu.make_async_copy(
              g_hbm.at[idx_v.at[sb, 0, k]], rowbuf.at[b], semg[b]).wait()

          @pl.when(k + 1 < SLAB)
          def _():
            pltpu.async_copy(
                g_hbm.at[idx_v.at[sb, 0, k + 1]], rowbuf.at[1 - b],
                semg[1 - b])

          pltpu.sync_copy(rowbuf.at[b], acc.at[idx_v.at[sb, 1, k]], add=True)
        return carry

      lax.fori_loop(0, SLAB // 2, chunks, 0)

    def outer(tt, carry):
      for sb in range(2):
        slab(tt * 2 + sb, sb)
      return carry

    lax.fori_loop(0, nslab // 2, outer, 0)
    plsc.subcore_barrier()

    pltpu.sync_copy(acc.at[pl.ds(s * rpt, rpt)],
                    out_hbm.at[c, pl.ds(s * rpt, rpt)])

  return agg_kernel


def _tc_pre(xp, W1, degp):
  npad = xp.shape[0]

  def body(x_ref, w_ref, dp_ref, g_ref, dinv_ref):
    deg = dp_ref[0, :, 0:1] + dp_ref[1, :, 0:1] + 1.0
    dinv = 1.0 / jnp.sqrt(deg)
    dinv_ref[...] = dinv
    g_ref[...] = jnp.dot(
        x_ref[...], w_ref[...], preferred_element_type=jnp.float32) * dinv

  return pl.pallas_call(
      body,
      grid=(npad // BR,),
      in_specs=[
          pl.BlockSpec((BR, D), lambda i: (i, 0)),
          pl.BlockSpec((D, D), lambda i: (0, 0)),
          pl.BlockSpec((NC, BR, DW), lambda i: (0, i, 0)),
      ],
      out_specs=[
          pl.BlockSpec((BR, D), lambda i: (i, 0)),
          pl.BlockSpec((BR, 1), lambda i: (i, 0)),
      ],
      out_shape=[
          jax.ShapeDtypeStruct((npad, D), jnp.float32),
          jax.ShapeDtypeStruct((npad, 1), jnp.float32),
      ],
  )(xp, W1, degp)


def _tc_mid(p, g, dinv, b, W):
  npad = g.shape[0]

  def body(p_ref, g_ref, dinv_ref, b_ref, w_ref, out_ref):
    t = (p_ref[0] + p_ref[1] + g_ref[...]) * dinv_ref[...] + b_ref[...]
    t = jnp.maximum(t, 0.0)
    out_ref[...] = jnp.dot(
        t, w_ref[...], preferred_element_type=jnp.float32) * dinv_ref[...]

  return pl.pallas_call(
      body,
      grid=(npad // BR,),
      in_specs=[
          pl.BlockSpec((NC, BR, D), lambda i: (0, i, 0)),
          pl.BlockSpec((BR, D), lambda i: (i, 0)),
          pl.BlockSpec((BR, 1), lambda i: (i, 0)),
          pl.BlockSpec((1, D), lambda i: (0, 0)),
          pl.BlockSpec((D, D), lambda i: (0, 0)),
      ],
      out_specs=pl.BlockSpec((BR, D), lambda i: (i, 0)),
      out_shape=jax.ShapeDtypeStruct((npad, D), jnp.float32),
  )(p, g, dinv, b, W)


def _tc_post(p, g, dinv, b):
  npad = g.shape[0]

  def body(p_ref, g_ref, dinv_ref, b_ref, out_ref):
    out_ref[...] = (
        (p_ref[0] + p_ref[1] + g_ref[...]) * dinv_ref[...] + b_ref[...])

  return pl.pallas_call(
      body,
      grid=(npad // BR,),
      in_specs=[
          pl.BlockSpec((NC, BR, D), lambda i: (0, i, 0)),
          pl.BlockSpec((BR, D), lambda i: (i, 0)),
          pl.BlockSpec((BR, 1), lambda i: (i, 0)),
          pl.BlockSpec((1, D), lambda i: (0, 0)),
      ],
      out_specs=pl.BlockSpec((BR, D), lambda i: (i, 0)),
      out_shape=jax.ShapeDtypeStruct((npad, D), jnp.float32),
  )(p, g, dinv, b)


def kernel(x, edge_index, W1, b1, W2, b2, W3, b3):
  N, d_in = x.shape
  E = edge_index.shape[1]

  npad = -(-(N + 1) // (NS * D)) * (NS * D)
  ch = -(-E // (NW * D * 2 * SLAB)) * (2 * SLAB)
  nslab = ch // SLAB
  epad = NW * ch * D

  src = edge_index[0].astype(jnp.int32)
  dst = edge_index[1].astype(jnp.int32)
  pad = jnp.full((epad - E,), N, jnp.int32)
  pad_dst = N + (jnp.arange(epad - E, dtype=jnp.int32) % (npad - N))
  src5 = jnp.concatenate([src, pad]).reshape(NW, nslab, 1, SLAB, D)
  dst5 = jnp.concatenate([dst, pad_dst]).reshape(NW, nslab, 1, SLAB, D)
  edges5 = jnp.concatenate([src5, dst5], axis=2)
  dst3 = dst5.reshape(NW, ch, D)
  xp = jnp.concatenate([x, jnp.zeros((npad - N, d_in), x.dtype)])

  degp = _make_deg_kernel(npad, ch)(dst3)

  agg = _make_agg_kernel(npad, nslab)

  g, dinv = _tc_pre(xp, W1, degp)
  p = agg(g, edges5)
  g = _tc_mid(p, g, dinv, b1.reshape(1, D), W2)
  p = agg(g, edges5)
  g = _tc_mid(p, g, dinv, b2.reshape(1, D), W3)
  p = agg(g, edges5)
  out = _tc_post(p, g, dinv, b3.reshape(1, D))
  return out[:N]


# speedup vs baseline: 3.0016x; 3.0006x over previous
import functools

import jax
import jax.numpy as jnp
from jax import lax
from jax.experimental import pallas as pl
from jax.experimental.pallas import tpu as pltpu
from jax.experimental.pallas import tpu_sc as plsc

NC = 2
NS = 16
NW = NC * NS
D = 128
SLAB = 8
BR = 1024
DW = 16


def _mesh():
  return plsc.VectorSubcoreMesh(
      core_axis_name="c", subcore_axis_name="s", num_cores=NC,
      num_subcores=NS)


def _make_deg_kernel(npad, ch):
  rpt = npad // NS

  @functools.partial(
      pl.kernel,
      out_type=jax.ShapeDtypeStruct((NC, npad, DW), jnp.float32),
      mesh=_mesh(),
      scratch_types=[
          pltpu.VMEM((ch, D), jnp.int32),
          pltpu.VMEM((D, DW), jnp.float32),
          pltpu.VMEM_SHARED((npad, DW), jnp.float32),
      ],
  )
  def deg_kernel(dst_hbm, out_hbm, dst_v, buf, shist):
    c = lax.axis_index("c")
    s = lax.axis_index("s")
    wid = s * NC + c

    def fill(j, val):
      buf[j, pl.ds(0, 16)] = jnp.full((16,), val, jnp.float32)
      return val

    lax.fori_loop(0, D, fill, 0.0)
    for k in range(rpt // D):
      pltpu.sync_copy(buf, shist.at[pl.ds(s * rpt + k * D, D)])
    lax.fori_loop(0, D, fill, 1.0)
    plsc.subcore_barrier()

    pltpu.sync_copy(dst_hbm.at[wid], dst_v)

    def body(j, carry):
      pltpu.sync_copy(buf, shist.at[dst_v.at[j]], add=True)
      return carry

    lax.fori_loop(0, ch, body, 0)
    plsc.subcore_barrier()

    pltpu.sync_copy(shist.at[pl.ds(s * rpt, rpt)],
                    out_hbm.at[c, pl.ds(s * rpt, rpt)])

  return deg_kernel


def _make_agg_kernel(npad, nslab):
  rpt = npad // NS

  @functools.partial(
      pl.kernel,
      out_type=jax.ShapeDtypeStruct((NC, npad, D), jnp.float32),
      mesh=_mesh(),
      scratch_types=[
          pltpu.VMEM((2, 2, SLAB, D), jnp.int32),
          pltpu.VMEM((2, D, D), jnp.float32),
          pltpu.VMEM_SHARED((npad, D), jnp.float32),
          pltpu.SemaphoreType.DMA,
          pltpu.SemaphoreType.DMA,
          pltpu.SemaphoreType.DMA,
          pltpu.SemaphoreType.DMA,
      ],
  )
  def agg_kernel(g_hbm, e_hbm, out_hbm, idx_v, rowbuf, acc,
                 semi0, semi1, semg0, semg1):
    c = lax.axis_index("c")
    s = lax.axis_index("s")
    wid = s * NC + c
    semi = (semi0, semi1)
    semg = (semg0, semg1)

    def zrow(j, carry):
      for k in range(D // 16):
        rowbuf[0, j, pl.ds(k * 16, 16)] = jnp.zeros((16,), jnp.float32)
      return carry

    lax.fori_loop(0, D, zrow, 0)
    for k in range(rpt // D):
      pltpu.sync_copy(rowbuf.at[0], acc.at[pl.ds(s * rpt + k * D, D)])
    plsc.subcore_barrier()

    pltpu.async_copy(e_hbm.at[wid, 0], idx_v.at[0], semi[0])

    def slab(t, sb):
      pltpu.make_async_copy(e_hbm.at[wid, t], idx_v.at[sb], semi[sb]).wait()

      @pl.when(t + 1 < nslab)
      def _():
        pltpu.async_copy(e_hbm.at[wid, t + 1], idx_v.at[1 - sb], semi[1 - sb])

      pltpu.async_copy(g_hbm.at[idx_v.at[sb, 0, 0]], rowbuf.at[0], semg[0])

      def chunks(kk, carry):
        for b in range(2):
          k = kk * 2 + b
          pltpu.make_async_copy(
              g_hbm.at[idx_v.at[sb, 0, k]], rowbuf.at[b], semg[b]).wait()

          @pl.when(k + 1 < SLAB)
          def _():
            pltpu.async_copy(
                g_hbm.at[idx_v.at[sb, 0, k + 1]], rowbuf.at[1 - b],
                semg[1 - b])

          pltpu.sync_copy(rowbuf.at[b], acc.at[idx_v.at[sb, 1, k]], add=True)
        return carry

      lax.fori_loop(0, SLAB // 2, chunks, 0)

    def outer(tt, carry):
      for sb in range(2):
        slab(tt * 2 + sb, sb)
      return carry

    lax.fori_loop(0, nslab // 2, outer, 0)
    plsc.subcore_barrier()

    pltpu.sync_copy(acc.at[pl.ds(s * rpt, rpt)],
                    out_hbm.at[c, pl.ds(s * rpt, rpt)])

  return agg_kernel


def _tc_pre(xp, W1, degp):
  npad = xp.shape[0]

  def body(x_ref, w_ref, dp_ref, g_ref, dinv_ref):
    deg = dp_ref[0, :, 0:1] + dp_ref[1, :, 0:1] + 1.0
    dinv = 1.0 / jnp.sqrt(deg)
    dinv_ref[...] = dinv
    g_ref[...] = jnp.dot(
        x_ref[...], w_ref[...], preferred_element_type=jnp.float32) * dinv

  return pl.pallas_call(
      body,
      grid=(npad // BR,),
      in_specs=[
          pl.BlockSpec((BR, D), lambda i: (i, 0)),
          pl.BlockSpec((D, D), lambda i: (0, 0)),
          pl.BlockSpec((NC, BR, DW), lambda i: (0, i, 0)),
      ],
      out_specs=[
          pl.BlockSpec((BR, D), lambda i: (i, 0)),
          pl.BlockSpec((BR, 1), lambda i: (i, 0)),
      ],
      out_shape=[
          jax.ShapeDtypeStruct((npad, D), jnp.float32),
          jax.ShapeDtypeStruct((npad, 1), jnp.float32),
      ],
  )(xp, W1, degp)


def _tc_mid(p, g, dinv, b, W):
  npad = g.shape[0]

  def body(p_ref, g_ref, dinv_ref, b_ref, w_ref, out_ref):
    t = (p_ref[0] + p_ref[1] + g_ref[...]) * dinv_ref[...] + b_ref[...]
    t = jnp.maximum(t, 0.0)
    out_ref[...] = jnp.dot(
        t, w_ref[...], preferred_element_type=jnp.float32) * dinv_ref[...]

  return pl.pallas_call(
      body,
      grid=(npad // BR,),
      in_specs=[
          pl.BlockSpec((NC, BR, D), lambda i: (0, i, 0)),
          pl.BlockSpec((BR, D), lambda i: (i, 0)),
          pl.BlockSpec((BR, 1), lambda i: (i, 0)),
          pl.BlockSpec((1, D), lambda i: (0, 0)),
          pl.BlockSpec((D, D), lambda i: (0, 0)),
      ],
      out_specs=pl.BlockSpec((BR, D), lambda i: (i, 0)),
      out_shape=jax.ShapeDtypeStruct((npad, D), jnp.float32),
  )(p, g, dinv, b, W)


def _tc_post(p, g, dinv, b):
  npad = g.shape[0]

  def body(p_ref, g_ref, dinv_ref, b_ref, out_ref):
    out_ref[...] = (
        (p_ref[0] + p_ref[1] + g_ref[...]) * dinv_ref[...] + b_ref[...])

  return pl.pallas_call(
      body,
      grid=(npad // BR,),
      in_specs=[
          pl.BlockSpec((NC, BR, D), lambda i: (0, i, 0)),
          pl.BlockSpec((BR, D), lambda i: (i, 0)),
          pl.BlockSpec((BR, 1), lambda i: (i, 0)),
          pl.BlockSpec((1, D), lambda i: (0, 0)),
      ],
      out_specs=pl.BlockSpec((BR, D), lambda i: (i, 0)),
      out_shape=jax.ShapeDtypeStruct((npad, D), jnp.float32),
  )(p, g, dinv, b)


def kernel(x, edge_index, W1, b1, W2, b2, W3, b3):
  N, d_in = x.shape
  E = edge_index.shape[1]

  npad = -(-(N + 1) // (NS * D)) * (NS * D)
  ch = -(-E // (NW * D * 2 * SLAB)) * (2 * SLAB)
  nslab = ch // SLAB
  epad = NW * ch * D

  src = edge_index[0].astype(jnp.int32)
  dst = edge_index[1].astype(jnp.int32)
  pad_ar = jnp.arange(epad - E, dtype=jnp.int32) % (npad - N)
  def deal(v, padv):
    c3 = jnp.concatenate([v, padv]).reshape(ch, NW, D).transpose(1, 0, 2)
    return c3.reshape(NW, nslab, 1, SLAB, D)

  src5 = deal(src, N + pad_ar)
  dst5 = deal(dst, N + pad_ar)
  edges5 = jnp.concatenate([src5, dst5], axis=2)
  dst3 = dst5.reshape(NW, ch, D)
  xp = jnp.concatenate([x, jnp.zeros((npad - N, d_in), x.dtype)])

  degp = _make_deg_kernel(npad, ch)(dst3)

  agg = _make_agg_kernel(npad, nslab)

  g, dinv = _tc_pre(xp, W1, degp)
  p = agg(g, edges5)
  g = _tc_mid(p, g, dinv, b1.reshape(1, D), W2)
  p = agg(g, edges5)
  g = _tc_mid(p, g, dinv, b2.reshape(1, D), W3)
  p = agg(g, edges5)
  out = _tc_post(p, g, dinv, b3.reshape(1, D))
  return out[:N]


# speedup vs baseline: 3.0114x; 1.0033x over previous
import functools

import jax
import jax.numpy as jnp
from jax import lax
from jax.experimental import pallas as pl
from jax.experimental.pallas import tpu as pltpu
from jax.experimental.pallas import tpu_sc as plsc

NC = 2
NS = 16
NW = NC * NS
D = 128
SLAB = 8
BR = 1024
DW = 16


def _mesh():
  return plsc.VectorSubcoreMesh(
      core_axis_name="c", subcore_axis_name="s", num_cores=NC,
      num_subcores=NS)


def _make_deg_kernel(npad, ch):
  rpt = npad // NS

  @functools.partial(
      pl.kernel,
      out_type=jax.ShapeDtypeStruct((NC, npad, DW), jnp.float32),
      mesh=_mesh(),
      scratch_types=[
          pltpu.VMEM((ch, D), jnp.int32),
          pltpu.VMEM((D, DW), jnp.float32),
          pltpu.VMEM_SHARED((npad, DW), jnp.float32),
      ],
  )
  def deg_kernel(dst_hbm, out_hbm, dst_v, buf, shist):
    c = lax.axis_index("c")
    s = lax.axis_index("s")
    wid = s * NC + c

    def fill(j, val):
      buf[j, pl.ds(0, 16)] = jnp.full((16,), val, jnp.float32)
      return val

    lax.fori_loop(0, D, fill, 0.0)
    for k in range(rpt // D):
      pltpu.sync_copy(buf, shist.at[pl.ds(s * rpt + k * D, D)])
    lax.fori_loop(0, D, fill, 1.0)
    plsc.subcore_barrier()

    pltpu.sync_copy(dst_hbm.at[wid], dst_v)

    def body(j, carry):
      pltpu.sync_copy(buf, shist.at[dst_v.at[j]], add=True)
      return carry

    lax.fori_loop(0, ch, body, 0)
    plsc.subcore_barrier()

    pltpu.sync_copy(shist.at[pl.ds(s * rpt, rpt)],
                    out_hbm.at[c, pl.ds(s * rpt, rpt)])

  return deg_kernel


def _make_agg_kernel(npad, nslab):
  rpt = npad // NS

  @functools.partial(
      pl.kernel,
      out_type=jax.ShapeDtypeStruct((NC, npad, D), jnp.float32),
      mesh=_mesh(),
      scratch_types=[
          pltpu.VMEM((2, 2, SLAB, D), jnp.int32),
          pltpu.VMEM((2, D, D), jnp.float32),
          pltpu.VMEM_SHARED((npad, D), jnp.float32),
          pltpu.SemaphoreType.DMA,
          pltpu.SemaphoreType.DMA,
          pltpu.SemaphoreType.DMA,
          pltpu.SemaphoreType.DMA,
      ],
  )
  def agg_kernel(g_hbm, e_hbm, out_hbm, idx_v, rowbuf, acc,
                 semi0, semi1, semg0, semg1):
    c = lax.axis_index("c")
    s = lax.axis_index("s")
    wid = s * NC + c
    semi = (semi0, semi1)
    semg = (semg0, semg1)

    def zrow(j, carry):
      for k in range(D // 16):
        rowbuf[0, j, pl.ds(k * 16, 16)] = jnp.zeros((16,), jnp.float32)
      return carry

    lax.fori_loop(0, D, zrow, 0)
    for k in range(rpt // D):
      pltpu.sync_copy(rowbuf.at[0], acc.at[pl.ds(s * rpt + k * D, D)])
    plsc.subcore_barrier()

    pltpu.async_copy(e_hbm.at[wid, 0], idx_v.at[0], semi[0])

    def slab(t, sb):
      pltpu.make_async_copy(e_hbm.at[wid, t], idx_v.at[sb], semi[sb]).wait()

      @pl.when(t + 1 < nslab)
      def _():
        pltpu.async_copy(e_hbm.at[wid, t + 1], idx_v.at[1 - sb], semi[1 - sb])

      pltpu.async_copy(g_hbm.at[idx_v.at[sb, 0, 0]], rowbuf.at[0], semg[0])

      def chunks(kk, carry):
        for b in range(2):
          k = kk * 2 + b
          pltpu.make_async_copy(
              g_hbm.at[idx_v.at[sb, 0, k]], rowbuf.at[b], semg[b]).wait()

          @pl.when(k + 1 < SLAB)
          def _():
            pltpu.async_copy(
                g_hbm.at[idx_v.at[sb, 0, k + 1]], rowbuf.at[1 - b],
                semg[1 - b])

          pltpu.sync_copy(rowbuf.at[b], acc.at[idx_v.at[sb, 1, k]], add=True)
        return carry

      lax.fori_loop(0, SLAB // 2, chunks, 0)

    def outer(tt, carry):
      for sb in range(2):
        slab(tt * 2 + sb, sb)
      return carry

    lax.fori_loop(0, nslab // 2, outer, 0)
    plsc.subcore_barrier()

    pltpu.sync_copy(acc.at[pl.ds(s * rpt, rpt)],
                    out_hbm.at[c, pl.ds(s * rpt, rpt)])

  return agg_kernel


def _tc_pre(xp, W1, degp):
  npad = xp.shape[0]

  def body(x_ref, w_ref, dp_ref, g_ref, dinv_ref):
    deg = dp_ref[0, :, 0:1] + dp_ref[1, :, 0:1] + 1.0
    dinv = 1.0 / jnp.sqrt(deg)
    dinv_ref[...] = dinv
    g_ref[...] = jnp.dot(
        x_ref[...], w_ref[...], preferred_element_type=jnp.float32) * dinv

  return pl.pallas_call(
      body,
      grid=(npad // BR,),
      in_specs=[
          pl.BlockSpec((BR, D), lambda i: (i, 0)),
          pl.BlockSpec((D, D), lambda i: (0, 0)),
          pl.BlockSpec((NC, BR, DW), lambda i: (0, i, 0)),
      ],
      out_specs=[
          pl.BlockSpec((BR, D), lambda i: (i, 0)),
          pl.BlockSpec((BR, 1), lambda i: (i, 0)),
      ],
      out_shape=[
          jax.ShapeDtypeStruct((npad, D), jnp.float32),
          jax.ShapeDtypeStruct((npad, 1), jnp.float32),
      ],
  )(xp, W1, degp)


def _tc_mid(p, g, dinv, b, W):
  npad = g.shape[0]

  def body(p_ref, g_ref, dinv_ref, b_ref, w_ref, out_ref):
    t = (p_ref[0] + p_ref[1] + g_ref[...]) * dinv_ref[...] + b_ref[...]
    t = jnp.maximum(t, 0.0)
    out_ref[...] = jnp.dot(
        t, w_ref[...], preferred_element_type=jnp.float32) * dinv_ref[...]

  return pl.pallas_call(
      body,
      grid=(npad // BR,),
      in_specs=[
          pl.BlockSpec((NC, BR, D), lambda i: (0, i, 0)),
          pl.BlockSpec((BR, D), lambda i: (i, 0)),
          pl.BlockSpec((BR, 1), lambda i: (i, 0)),
          pl.BlockSpec((1, D), lambda i: (0, 0)),
          pl.BlockSpec((D, D), lambda i: (0, 0)),
      ],
      out_specs=pl.BlockSpec((BR, D), lambda i: (i, 0)),
      out_shape=jax.ShapeDtypeStruct((npad, D), jnp.float32),
  )(p, g, dinv, b, W)


def _tc_post(p, g, dinv, b):
  npad = g.shape[0]

  def body(p_ref, g_ref, dinv_ref, b_ref, out_ref):
    out_ref[...] = (
        (p_ref[0] + p_ref[1] + g_ref[...]) * dinv_ref[...] + b_ref[...])

  return pl.pallas_call(
      body,
      grid=(npad // BR,),
      in_specs=[
          pl.BlockSpec((NC, BR, D), lambda i: (0, i, 0)),
          pl.BlockSpec((BR, D), lambda i: (i, 0)),
          pl.BlockSpec((BR, 1), lambda i: (i, 0)),
          pl.BlockSpec((1, D), lambda i: (0, 0)),
      ],
      out_specs=pl.BlockSpec((BR, D), lambda i: (i, 0)),
      out_shape=jax.ShapeDtypeStruct((npad, D), jnp.float32),
  )(p, g, dinv, b)


def kernel(x, edge_index, W1, b1, W2, b2, W3, b3):
  N, d_in = x.shape
  E = edge_index.shape[1]

  npad = -(-(N + 1) // (NS * D)) * (NS * D)
  ch = -(-E // (NW * D * 2 * SLAB)) * (2 * SLAB)
  nslab = ch // SLAB
  epad = NW * ch * D

  src = edge_index[0].astype(jnp.int32)
  dst = edge_index[1].astype(jnp.int32)
  pad_ar = N + jnp.arange(epad - E, dtype=jnp.int32) % (npad - N)
  src5 = jnp.concatenate([src, pad_ar]).reshape(NW, nslab, 1, SLAB, D)
  dst5 = jnp.concatenate([dst, pad_ar]).reshape(NW, nslab, 1, SLAB, D)
  edges5 = jnp.concatenate([src5, dst5], axis=2)
  dst3 = dst5.reshape(NW, ch, D)
  xp = jnp.concatenate([x, jnp.zeros((npad - N, d_in), x.dtype)])

  degp = _make_deg_kernel(npad, ch)(dst3)

  agg = _make_agg_kernel(npad, nslab)

  g, dinv = _tc_pre(xp, W1, degp)
  p = agg(g, edges5)
  g = _tc_mid(p, g, dinv, b1.reshape(1, D), W2)
  p = agg(g, edges5)
  g = _tc_mid(p, g, dinv, b2.reshape(1, D), W3)
  p = agg(g, edges5)
  out = _tc_post(p, g, dinv, b3.reshape(1, D))
  return out[:N]
